# baseline (speedup 1.0000x reference)
import functools

import jax
import jax.numpy as jnp
from jax import lax
from jax.experimental import pallas as pl
from jax.experimental.pallas import tpu as pltpu
from jax.experimental.pallas import tpu_sc as plsc

N_ENTITY = 100000
N_NEIGHBOR = 16
BATCH = 16384

NC = 2
NS = 16
NW = NC * NS
SPW = BATCH // NW
WIDE = N_NEIGHBOR * N_NEIGHBOR

EPW = 3136
ECH = 1568
ELAST = N_ENTITY - EPW


def _tr_body(entt, relt, ent_rm, rel_rm, chunk_v, rowbuf_v, sem):
    wid = lax.axis_index("s") * NC + lax.axis_index("c")
    start = jnp.minimum(wid * EPW, ELAST)
    rows16 = lax.iota(jnp.int32, 16)

    for src, dst in ((entt, ent_rm), (relt, rel_rm)):
        for c in range(EPW // ECH):
            e0 = start + c * ECH
            pltpu.async_copy(src.at[:, pl.ds(e0, ECH)], chunk_v, sem).wait()

            def body(e, _):
                v = plsc.load_gather(chunk_v, [rows16, jnp.full((16,), 0, jnp.int32) + e])
                rowbuf_v[e] = v
                return 0

            lax.fori_loop(0, ECH, body, 0, unroll=8)
            pltpu.sync_copy(rowbuf_v, dst.at[pl.ds(e0, ECH)])


def _rf_body(x_hbm, ent_hbm, rel_hbm,
             out1t, out2, out3, out4,
             idx0_v, ent1_v, rel1_v, ent1t_v, ent2_v, rel2_v,
             sem_h1e, sem_h1r, sem_e0, sem_e1, sem_r0, sem_r1):
    wid = lax.axis_index("s") * NC + lax.axis_index("c")
    base = wid * SPW

    pltpu.sync_copy(x_hbm.at[pl.ds(base, SPW)], idx0_v)

    cp_e1 = pltpu.async_copy(ent_hbm.at[idx0_v], ent1_v, sem_h1e)
    cp_r1 = pltpu.async_copy(rel_hbm.at[idx0_v], rel1_v, sem_h1r)
    cp_e1.wait()

    rows16 = lax.iota(jnp.int32, 16)

    def transpose_col(k):
        col = jnp.full((16,), k, jnp.int32)
        for g in range(SPW // 16):
            v = plsc.load_gather(ent1_v, [rows16 + (g * 16), col])
            ent1t_v[k, pl.ds(g * 16, 16)] = v

    transpose_col(0)

    sem_e = (sem_e0, sem_e1)
    sem_r = (sem_r0, sem_r1)
    cp_e = [None, None]
    cp_r = [None, None]

    for k in range(N_NEIGHBOR + 1):
        if k < N_NEIGHBOR:
            b = k % 2
            idx_k = ent1t_v.at[k]
            cp_e[b] = pltpu.async_copy(ent_hbm.at[idx_k], ent2_v.at[b], sem_e[b])
            cp_r[b] = pltpu.async_copy(rel_hbm.at[idx_k], rel2_v.at[b], sem_r[b])
        if k + 1 < N_NEIGHBOR:
            transpose_col(k + 1)
        if k == N_NEIGHBOR - 1:
            pltpu.sync_copy(ent1t_v, out1t.at[:, pl.ds(base, SPW)])
        if k == 0:
            cp_r1.wait()
            pltpu.sync_copy(rel1_v, out3.at[pl.ds(base, SPW)])
        else:
            pb = (k - 1) % 2
            cols = pl.ds((k - 1) * N_NEIGHBOR, N_NEIGHBOR)
            cp_e[pb].wait()
            pltpu.sync_copy(ent2_v.at[pb], out2.at[pl.ds(base, SPW), cols])
            cp_r[pb].wait()
            pltpu.sync_copy(rel2_v.at[pb], out4.at[pl.ds(base, SPW), cols])


@jax.jit
def kernel(x, adj_entity, adj_relation):
    i32 = jnp.int32
    x_flat = x.reshape(BATCH).astype(i32)
    entt = adj_entity.astype(i32).T
    relt = adj_relation.astype(i32).T

    mesh = plsc.VectorSubcoreMesh(core_axis_name="c", subcore_axis_name="s")
    params = pltpu.CompilerParams(
        use_tc_tiling_on_sc=False, needs_layout_passes=False)

    relayout = pl.kernel(
        _tr_body,
        out_type=(
            jax.ShapeDtypeStruct((N_ENTITY, N_NEIGHBOR), i32),
            jax.ShapeDtypeStruct((N_ENTITY, N_NEIGHBOR), i32),
        ),
        mesh=mesh,
        compiler_params=params,
        scratch_types=[
            pltpu.VMEM((N_NEIGHBOR, ECH), i32),
            pltpu.VMEM((ECH, N_NEIGHBOR), i32),
            pltpu.SemaphoreType.DMA,
        ],
    )
    ent_rm, rel_rm = relayout(entt, relt)

    expand = pl.kernel(
        _rf_body,
        out_type=(
            jax.ShapeDtypeStruct((N_NEIGHBOR, BATCH), i32),
            jax.ShapeDtypeStruct((BATCH, WIDE), i32),
            jax.ShapeDtypeStruct((BATCH, N_NEIGHBOR), i32),
            jax.ShapeDtypeStruct((BATCH, WIDE), i32),
        ),
        mesh=mesh,
        compiler_params=params,
        scratch_types=[
            pltpu.VMEM((SPW,), i32),
            pltpu.VMEM((SPW, N_NEIGHBOR), i32),
            pltpu.VMEM((SPW, N_NEIGHBOR), i32),
            pltpu.VMEM((N_NEIGHBOR, SPW), i32),
            pltpu.VMEM((2, SPW, N_NEIGHBOR), i32),
            pltpu.VMEM((2, SPW, N_NEIGHBOR), i32),
            pltpu.SemaphoreType.DMA,
            pltpu.SemaphoreType.DMA,
            pltpu.SemaphoreType.DMA,
            pltpu.SemaphoreType.DMA,
            pltpu.SemaphoreType.DMA,
            pltpu.SemaphoreType.DMA,
        ],
    )
    ent1t, ent2, rel1, rel2 = expand(x_flat, ent_rm, rel_rm)
    return (x, ent1t.T, ent2, rel1, rel2)


# speedup vs baseline: 1.1752x; 1.1752x over previous
import functools

import jax
import jax.numpy as jnp
from jax import lax
from jax.experimental import pallas as pl
from jax.experimental.pallas import tpu as pltpu
from jax.experimental.pallas import tpu_sc as plsc

N_ENTITY = 100000
N_NEIGHBOR = 16
BATCH = 16384

NC = 2
NS = 16
NW = NC * NS
SPW = BATCH // NW
WIDE = N_NEIGHBOR * N_NEIGHBOR

EPW = 3136
ECH = 1568
ELAST = N_ENTITY - EPW


def _tr_body(entt, relt, ent_rm, rel_rm, chunk_v, rowbuf_v, sem):
    wid = lax.axis_index("s") * NC + lax.axis_index("c")
    start = jnp.minimum(wid * EPW, ELAST)
    rows16 = lax.iota(jnp.int32, 16)

    for src, dst in ((entt, ent_rm), (relt, rel_rm)):
        for c in range(EPW // ECH):
            e0 = start + c * ECH
            pltpu.async_copy(src.at[:, pl.ds(e0, ECH)], chunk_v, sem).wait()

            @plsc.parallel_loop(0, ECH, unroll=16)
            def body(e):
                v = plsc.load_gather(chunk_v, [rows16, lax.broadcast(e, (16,))])
                rowbuf_v[e] = v

            pltpu.sync_copy(rowbuf_v, dst.at[pl.ds(e0, ECH)])


def _rf_body(x_hbm, ent_hbm, rel_hbm,
             out1t, out2, out3, out4,
             idx0_v, ent1_v, rel1_v, ent1t_v, ent2_v, rel2_v,
             sem_h1e, sem_h1r, sem_e0, sem_e1, sem_r0, sem_r1):
    wid = lax.axis_index("s") * NC + lax.axis_index("c")
    base = wid * SPW

    pltpu.sync_copy(x_hbm.at[pl.ds(base, SPW)], idx0_v)

    cp_e1 = pltpu.async_copy(ent_hbm.at[idx0_v], ent1_v, sem_h1e)
    cp_r1 = pltpu.async_copy(rel_hbm.at[idx0_v], rel1_v, sem_h1r)
    cp_e1.wait()

    rows16 = lax.iota(jnp.int32, 16)

    def transpose_col(k):
        col = jnp.full((16,), k, jnp.int32)
        for g in range(SPW // 16):
            v = plsc.load_gather(ent1_v, [rows16 + (g * 16), col])
            ent1t_v[k, pl.ds(g * 16, 16)] = v

    transpose_col(0)

    sem_e = (sem_e0, sem_e1)
    sem_r = (sem_r0, sem_r1)
    cp_e = [None, None]
    cp_r = [None, None]

    for k in range(N_NEIGHBOR + 1):
        if k < N_NEIGHBOR:
            b = k % 2
            idx_k = ent1t_v.at[k]
            cp_e[b] = pltpu.async_copy(ent_hbm.at[idx_k], ent2_v.at[b], sem_e[b])
            cp_r[b] = pltpu.async_copy(rel_hbm.at[idx_k], rel2_v.at[b], sem_r[b])
        if k + 1 < N_NEIGHBOR:
            transpose_col(k + 1)
        if k == N_NEIGHBOR - 1:
            pltpu.sync_copy(ent1t_v, out1t.at[:, pl.ds(base, SPW)])
        if k == 0:
            cp_r1.wait()
            pltpu.sync_copy(rel1_v, out3.at[pl.ds(base, SPW)])
        else:
            pb = (k - 1) % 2
            cols = pl.ds((k - 1) * N_NEIGHBOR, N_NEIGHBOR)
            cp_e[pb].wait()
            pltpu.sync_copy(ent2_v.at[pb], out2.at[pl.ds(base, SPW), cols])
            cp_r[pb].wait()
            pltpu.sync_copy(rel2_v.at[pb], out4.at[pl.ds(base, SPW), cols])


@jax.jit
def kernel(x, adj_entity, adj_relation):
    i32 = jnp.int32
    x_flat = x.reshape(BATCH).astype(i32)
    entt = adj_entity.astype(i32).T
    relt = adj_relation.astype(i32).T

    mesh = plsc.VectorSubcoreMesh(core_axis_name="c", subcore_axis_name="s")
    params = pltpu.CompilerParams(
        use_tc_tiling_on_sc=False, needs_layout_passes=False)

    relayout = pl.kernel(
        _tr_body,
        out_type=(
            jax.ShapeDtypeStruct((N_ENTITY, N_NEIGHBOR), i32),
            jax.ShapeDtypeStruct((N_ENTITY, N_NEIGHBOR), i32),
        ),
        mesh=mesh,
        compiler_params=params,
        scratch_types=[
            pltpu.VMEM((N_NEIGHBOR, ECH), i32),
            pltpu.VMEM((ECH, N_NEIGHBOR), i32),
            pltpu.SemaphoreType.DMA,
        ],
    )
    ent_rm, rel_rm = relayout(entt, relt)

    expand = pl.kernel(
        _rf_body,
        out_type=(
            jax.ShapeDtypeStruct((N_NEIGHBOR, BATCH), i32),
            jax.ShapeDtypeStruct((BATCH, WIDE), i32),
            jax.ShapeDtypeStruct((BATCH, N_NEIGHBOR), i32),
            jax.ShapeDtypeStruct((BATCH, WIDE), i32),
        ),
        mesh=mesh,
        compiler_params=params,
        scratch_types=[
            pltpu.VMEM((SPW,), i32),
            pltpu.VMEM((SPW, N_NEIGHBOR), i32),
            pltpu.VMEM((SPW, N_NEIGHBOR), i32),
            pltpu.VMEM((N_NEIGHBOR, SPW), i32),
            pltpu.VMEM((2, SPW, N_NEIGHBOR), i32),
            pltpu.VMEM((2, SPW, N_NEIGHBOR), i32),
            pltpu.SemaphoreType.DMA,
            pltpu.SemaphoreType.DMA,
            pltpu.SemaphoreType.DMA,
            pltpu.SemaphoreType.DMA,
            pltpu.SemaphoreType.DMA,
            pltpu.SemaphoreType.DMA,
        ],
    )
    ent1t, ent2, rel1, rel2 = expand(x_flat, ent_rm, rel_rm)
    return (x, ent1t.T, ent2, rel1, rel2)


# speedup vs baseline: 1.1853x; 1.0086x over previous
import functools

import jax
import jax.numpy as jnp
from jax import lax
from jax.experimental import pallas as pl
from jax.experimental.pallas import tpu as pltpu
from jax.experimental.pallas import tpu_sc as plsc

N_ENTITY = 100000
N_NEIGHBOR = 16
BATCH = 16384

NC = 2
NS = 16
NW = NC * NS
SPW = BATCH // NW
WIDE = N_NEIGHBOR * N_NEIGHBOR

EPW = 3136
ECH = 1568
ELAST = N_ENTITY - EPW


def _tr_body(entt, relt, ent_rm, rel_rm, chunk_v, rowbuf_v, sem):
    wid = lax.axis_index("s") * NC + lax.axis_index("c")
    start = jnp.minimum(wid * EPW, ELAST)
    colpos = lax.iota(jnp.int32, 16) * ECH

    for src, dst in ((entt, ent_rm), (relt, rel_rm)):
        for c in range(EPW // ECH):
            e0 = start + c * ECH
            cps = [
                pltpu.async_copy(src.at[k, pl.ds(e0, ECH)],
                                 chunk_v.at[pl.ds(k * ECH, ECH)], sem)
                for k in range(N_NEIGHBOR)
            ]
            for cp in cps:
                cp.wait()

            @plsc.parallel_loop(0, ECH, unroll=16)
            def body(e):
                rowbuf_v[e] = plsc.load_gather(chunk_v, [colpos + e])

            pltpu.sync_copy(rowbuf_v, dst.at[pl.ds(e0, ECH)])


def _rf_body(x_hbm, ent_hbm, rel_hbm,
             out1t, out2, out3, out4,
             idx0_v, ent1_v, rel1_v, ent1t_v, ent2_v, rel2_v,
             sem_h1e, sem_h1r, sem_e0, sem_e1, sem_r0, sem_r1):
    wid = lax.axis_index("s") * NC + lax.axis_index("c")
    base = wid * SPW

    pltpu.sync_copy(x_hbm.at[pl.ds(base, SPW)], idx0_v)

    cp_e1 = pltpu.async_copy(ent_hbm.at[idx0_v], ent1_v, sem_h1e)
    cp_r1 = pltpu.async_copy(rel_hbm.at[idx0_v], rel1_v, sem_h1r)
    cp_e1.wait()

    rows16 = lax.iota(jnp.int32, 16)

    def transpose_col(k):
        col = jnp.full((16,), k, jnp.int32)
        for g in range(SPW // 16):
            v = plsc.load_gather(ent1_v, [rows16 + (g * 16), col])
            ent1t_v[k, pl.ds(g * 16, 16)] = v

    transpose_col(0)

    sem_e = (sem_e0, sem_e1)
    sem_r = (sem_r0, sem_r1)
    cp_e = [None, None]
    cp_r = [None, None]

    for k in range(N_NEIGHBOR + 1):
        if k < N_NEIGHBOR:
            b = k % 2
            idx_k = ent1t_v.at[k]
            cp_e[b] = pltpu.async_copy(ent_hbm.at[idx_k], ent2_v.at[b], sem_e[b])
            cp_r[b] = pltpu.async_copy(rel_hbm.at[idx_k], rel2_v.at[b], sem_r[b])
        if k + 1 < N_NEIGHBOR:
            transpose_col(k + 1)
        if k == N_NEIGHBOR - 1:
            pltpu.sync_copy(ent1t_v, out1t.at[:, pl.ds(base, SPW)])
        if k == 0:
            cp_r1.wait()
            pltpu.sync_copy(rel1_v, out3.at[pl.ds(base, SPW)])
        else:
            pb = (k - 1) % 2
            cols = pl.ds((k - 1) * N_NEIGHBOR, N_NEIGHBOR)
            cp_e[pb].wait()
            pltpu.sync_copy(ent2_v.at[pb], out2.at[pl.ds(base, SPW), cols])
            cp_r[pb].wait()
            pltpu.sync_copy(rel2_v.at[pb], out4.at[pl.ds(base, SPW), cols])


@jax.jit
def kernel(x, adj_entity, adj_relation):
    i32 = jnp.int32
    x_flat = x.reshape(BATCH).astype(i32)
    entt = adj_entity.astype(i32).T
    relt = adj_relation.astype(i32).T

    mesh = plsc.VectorSubcoreMesh(core_axis_name="c", subcore_axis_name="s")
    params = pltpu.CompilerParams(
        use_tc_tiling_on_sc=False, needs_layout_passes=False)

    relayout = pl.kernel(
        _tr_body,
        out_type=(
            jax.ShapeDtypeStruct((N_ENTITY, N_NEIGHBOR), i32),
            jax.ShapeDtypeStruct((N_ENTITY, N_NEIGHBOR), i32),
        ),
        mesh=mesh,
        compiler_params=params,
        scratch_types=[
            pltpu.VMEM((N_NEIGHBOR * ECH,), i32),
            pltpu.VMEM((ECH, N_NEIGHBOR), i32),
            pltpu.SemaphoreType.DMA,
        ],
    )
    ent_rm, rel_rm = relayout(entt, relt)

    expand = pl.kernel(
        _rf_body,
        out_type=(
            jax.ShapeDtypeStruct((N_NEIGHBOR, BATCH), i32),
            jax.ShapeDtypeStruct((BATCH, WIDE), i32),
            jax.ShapeDtypeStruct((BATCH, N_NEIGHBOR), i32),
            jax.ShapeDtypeStruct((BATCH, WIDE), i32),
        ),
        mesh=mesh,
        compiler_params=params,
        scratch_types=[
            pltpu.VMEM((SPW,), i32),
            pltpu.VMEM((SPW, N_NEIGHBOR), i32),
            pltpu.VMEM((SPW, N_NEIGHBOR), i32),
            pltpu.VMEM((N_NEIGHBOR, SPW), i32),
            pltpu.VMEM((2, SPW, N_NEIGHBOR), i32),
            pltpu.VMEM((2, SPW, N_NEIGHBOR), i32),
            pltpu.SemaphoreType.DMA,
            pltpu.SemaphoreType.DMA,
            pltpu.SemaphoreType.DMA,
            pltpu.SemaphoreType.DMA,
            pltpu.SemaphoreType.DMA,
            pltpu.SemaphoreType.DMA,
        ],
    )
    ent1t, ent2, rel1, rel2 = expand(x_flat, ent_rm, rel_rm)
    return (x, ent1t.T, ent2, rel1, rel2)


# speedup vs baseline: 1.2139x; 1.0241x over previous
import functools

import jax
import jax.numpy as jnp
from jax import lax
from jax.experimental import pallas as pl
from jax.experimental.pallas import tpu as pltpu
from jax.experimental.pallas import tpu_sc as plsc

N_ENTITY = 100000
N_NEIGHBOR = 16
BATCH = 16384

NC = 2
NS = 16
NW = NC * NS
SPW = BATCH // NW
WIDE = N_NEIGHBOR * N_NEIGHBOR

EPW = 3136
ECH = 1568
ELAST = N_ENTITY - EPW


def _tr_body(entt, relt, ent_rm, rel_rm, chunk_v, rowbuf_v, sem):
    wid = lax.axis_index("s") * NC + lax.axis_index("c")
    start = jnp.minimum(wid * EPW, ELAST)
    colpos = lax.iota(jnp.int32, 16) * ECH

    for src, dst in ((entt, ent_rm), (relt, rel_rm)):
        for c in range(EPW // ECH):
            e0 = start + c * ECH
            cps = [
                pltpu.async_copy(src.at[k, pl.ds(e0, ECH)],
                                 chunk_v.at[pl.ds(k * ECH, ECH)], sem)
                for k in range(N_NEIGHBOR)
            ]
            for cp in cps:
                cp.wait()

            @plsc.parallel_loop(0, ECH, unroll=32)
            def body(e):
                rowbuf_v[e] = plsc.load_gather(chunk_v, [colpos + e])

            pltpu.sync_copy(rowbuf_v, dst.at[pl.ds(e0, ECH)])


def _rf_body(x_hbm, ent_hbm, rel_hbm,
             out1t, out2, out3t, out4,
             idx0_v, ent1_v, rel1_v, ent1t_v, rel1t_v, ent2_v, rel2_v,
             sem_h1e, sem_h1r, sem_e0, sem_e1, sem_r0, sem_r1):
    wid = lax.axis_index("s") * NC + lax.axis_index("c")
    base = wid * SPW

    pltpu.sync_copy(x_hbm.at[pl.ds(base, SPW)], idx0_v)

    cp_e1 = pltpu.async_copy(ent_hbm.at[idx0_v], ent1_v, sem_h1e)
    cp_r1 = pltpu.async_copy(rel_hbm.at[idx0_v], rel1_v, sem_h1r)
    cp_e1.wait()

    rows16 = lax.iota(jnp.int32, 16)

    def transpose_col(k, src_v=None, dst_v=None):
        src_v = ent1_v if src_v is None else src_v
        dst_v = ent1t_v if dst_v is None else dst_v
        col = jnp.full((16,), k, jnp.int32)
        for g in range(SPW // 16):
            v = plsc.load_gather(src_v, [rows16 + (g * 16), col])
            dst_v[k, pl.ds(g * 16, 16)] = v

    transpose_col(0)

    sem_e = (sem_e0, sem_e1)
    sem_r = (sem_r0, sem_r1)
    cp_e = [None, None]
    cp_r = [None, None]

    for k in range(N_NEIGHBOR + 1):
        if k < N_NEIGHBOR:
            b = k % 2
            idx_k = ent1t_v.at[k]
            cp_e[b] = pltpu.async_copy(ent_hbm.at[idx_k], ent2_v.at[b], sem_e[b])
            cp_r[b] = pltpu.async_copy(rel_hbm.at[idx_k], rel2_v.at[b], sem_r[b])
        if k + 1 < N_NEIGHBOR:
            transpose_col(k + 1)
        if k == N_NEIGHBOR - 1:
            pltpu.sync_copy(ent1t_v, out1t.at[:, pl.ds(base, SPW)])
        if k == 0:
            cp_r1.wait()
            for kk in range(N_NEIGHBOR):
                transpose_col(kk, rel1_v, rel1t_v)
            pltpu.sync_copy(rel1t_v, out3t.at[:, pl.ds(base, SPW)])
        else:
            pb = (k - 1) % 2
            cols = pl.ds((k - 1) * N_NEIGHBOR, N_NEIGHBOR)
            cp_e[pb].wait()
            pltpu.sync_copy(ent2_v.at[pb], out2.at[pl.ds(base, SPW), cols])
            cp_r[pb].wait()
            pltpu.sync_copy(rel2_v.at[pb], out4.at[pl.ds(base, SPW), cols])


@jax.jit
def kernel(x, adj_entity, adj_relation):
    i32 = jnp.int32
    x_flat = x.reshape(BATCH).astype(i32)
    entt = adj_entity.astype(i32).T
    relt = adj_relation.astype(i32).T

    mesh = plsc.VectorSubcoreMesh(core_axis_name="c", subcore_axis_name="s")
    params = pltpu.CompilerParams(
        use_tc_tiling_on_sc=False, needs_layout_passes=False)

    relayout = pl.kernel(
        _tr_body,
        out_type=(
            jax.ShapeDtypeStruct((N_ENTITY, N_NEIGHBOR), i32),
            jax.ShapeDtypeStruct((N_ENTITY, N_NEIGHBOR), i32),
        ),
        mesh=mesh,
        compiler_params=params,
        scratch_types=[
            pltpu.VMEM((N_NEIGHBOR * ECH,), i32),
            pltpu.VMEM((ECH, N_NEIGHBOR), i32),
            pltpu.SemaphoreType.DMA,
        ],
    )
    ent_rm, rel_rm = relayout(entt, relt)

    expand = pl.kernel(
        _rf_body,
        out_type=(
            jax.ShapeDtypeStruct((N_NEIGHBOR, BATCH), i32),
            jax.ShapeDtypeStruct((BATCH, WIDE), i32),
            jax.ShapeDtypeStruct((N_NEIGHBOR, BATCH), i32),
            jax.ShapeDtypeStruct((BATCH, WIDE), i32),
        ),
        mesh=mesh,
        compiler_params=params,
        scratch_types=[
            pltpu.VMEM((SPW,), i32),
            pltpu.VMEM((SPW, N_NEIGHBOR), i32),
            pltpu.VMEM((SPW, N_NEIGHBOR), i32),
            pltpu.VMEM((N_NEIGHBOR, SPW), i32),
            pltpu.VMEM((N_NEIGHBOR, SPW), i32),
            pltpu.VMEM((2, SPW, N_NEIGHBOR), i32),
            pltpu.VMEM((2, SPW, N_NEIGHBOR), i32),
            pltpu.SemaphoreType.DMA,
            pltpu.SemaphoreType.DMA,
            pltpu.SemaphoreType.DMA,
            pltpu.SemaphoreType.DMA,
            pltpu.SemaphoreType.DMA,
            pltpu.SemaphoreType.DMA,
        ],
    )
    ent1t, ent2, rel1t, rel2 = expand(x_flat, ent_rm, rel_rm)
    return (x, ent1t.T, ent2, rel1t.T, rel2)


# speedup vs baseline: 1.2682x; 1.0448x over previous
import functools

import jax
import jax.numpy as jnp
from jax import lax
from jax.experimental import pallas as pl
from jax.experimental.pallas import tpu as pltpu
from jax.experimental.pallas import tpu_sc as plsc

N_ENTITY = 100000
N_NEIGHBOR = 16
BATCH = 16384

NC = 2
NS = 16
NW = NC * NS
SPW = BATCH // NW
WIDE = N_NEIGHBOR * N_NEIGHBOR

EPW = 3136
ECH = 1568
ELAST = N_ENTITY - EPW


def _tr_body(entt, relt, ent_rm, rel_rm, chunk_v, rowbuf_v,
             sem_r0, sem_r1, sem_w0, sem_w1):
    wid = lax.axis_index("s") * NC + lax.axis_index("c")
    start = jnp.minimum(wid * EPW, ELAST)
    colpos = lax.iota(jnp.int32, 16) * ECH

    units = [(src, dst, c)
             for src, dst in ((entt, ent_rm), (relt, rel_rm))
             for c in range(EPW // ECH)]
    sem_r = (sem_r0, sem_r1)
    sem_w = (sem_w0, sem_w1)
    rd = [None, None]
    wr = [None, None]

    def fire_reads(u, b):
        src = units[u][0]
        e0 = start + units[u][2] * ECH
        rd[b] = [
            pltpu.async_copy(src.at[k, pl.ds(e0, ECH)],
                             chunk_v.at[b, pl.ds(k * ECH, ECH)], sem_r[b])
            for k in range(N_NEIGHBOR)
        ]

    fire_reads(0, 0)
    for u in range(len(units)):
        b = u % 2
        if u + 1 < len(units):
            fire_reads(u + 1, (u + 1) % 2)
        for cp in rd[b]:
            cp.wait()
        if wr[b] is not None:
            wr[b].wait()

        chunk_b = chunk_v.at[b]
        rowbuf_b = rowbuf_v.at[b]

        @plsc.parallel_loop(0, ECH, unroll=32)
        def body(e):
            rowbuf_b[e] = plsc.load_gather(chunk_b, [colpos + e])

        dst = units[u][1]
        e0 = start + units[u][2] * ECH
        wr[b] = pltpu.async_copy(rowbuf_v.at[b], dst.at[pl.ds(e0, ECH)], sem_w[b])

    for cp in wr:
        if cp is not None:
            cp.wait()


def _rf_body(x_hbm, ent_hbm, rel_hbm,
             out1t, out2, out3t, out4,
             idx0_v, ent1_v, rel1_v, ent1t_v, rel1t_v, ent2_v, rel2_v,
             sem_h1e, sem_h1r, sem_e0, sem_e1, sem_r0, sem_r1):
    wid = lax.axis_index("s") * NC + lax.axis_index("c")
    base = wid * SPW

    pltpu.sync_copy(x_hbm.at[pl.ds(base, SPW)], idx0_v)

    cp_e1 = pltpu.async_copy(ent_hbm.at[idx0_v], ent1_v, sem_h1e)
    cp_r1 = pltpu.async_copy(rel_hbm.at[idx0_v], rel1_v, sem_h1r)
    cp_e1.wait()

    rows16 = lax.iota(jnp.int32, 16)

    def transpose_col(k, src_v=None, dst_v=None):
        src_v = ent1_v if src_v is None else src_v
        dst_v = ent1t_v if dst_v is None else dst_v
        col = jnp.full((16,), k, jnp.int32)
        for g in range(SPW // 16):
            v = plsc.load_gather(src_v, [rows16 + (g * 16), col])
            dst_v[k, pl.ds(g * 16, 16)] = v

    transpose_col(0)

    sem_e = (sem_e0, sem_e1)
    sem_r = (sem_r0, sem_r1)
    cp_e = [None, None]
    cp_r = [None, None]

    for k in range(N_NEIGHBOR + 1):
        if k < N_NEIGHBOR:
            b = k % 2
            idx_k = ent1t_v.at[k]
            cp_e[b] = pltpu.async_copy(ent_hbm.at[idx_k], ent2_v.at[b], sem_e[b])
            cp_r[b] = pltpu.async_copy(rel_hbm.at[idx_k], rel2_v.at[b], sem_r[b])
        if k + 1 < N_NEIGHBOR:
            transpose_col(k + 1)
        if k == N_NEIGHBOR - 1:
            pltpu.sync_copy(ent1t_v, out1t.at[:, pl.ds(base, SPW)])
        if k == 0:
            cp_r1.wait()
            for kk in range(N_NEIGHBOR):
                transpose_col(kk, rel1_v, rel1t_v)
            pltpu.sync_copy(rel1t_v, out3t.at[:, pl.ds(base, SPW)])
        else:
            pb = (k - 1) % 2
            cols = pl.ds((k - 1) * N_NEIGHBOR, N_NEIGHBOR)
            cp_e[pb].wait()
            pltpu.sync_copy(ent2_v.at[pb], out2.at[pl.ds(base, SPW), cols])
            cp_r[pb].wait()
            pltpu.sync_copy(rel2_v.at[pb], out4.at[pl.ds(base, SPW), cols])


@jax.jit
def kernel(x, adj_entity, adj_relation):
    i32 = jnp.int32
    x_flat = x.reshape(BATCH).astype(i32)
    entt = adj_entity.astype(i32).T
    relt = adj_relation.astype(i32).T

    mesh = plsc.VectorSubcoreMesh(core_axis_name="c", subcore_axis_name="s")
    params = pltpu.CompilerParams(
        use_tc_tiling_on_sc=False, needs_layout_passes=False)

    relayout = pl.kernel(
        _tr_body,
        out_type=(
            jax.ShapeDtypeStruct((N_ENTITY, N_NEIGHBOR), i32),
            jax.ShapeDtypeStruct((N_ENTITY, N_NEIGHBOR), i32),
        ),
        mesh=mesh,
        compiler_params=params,
        scratch_types=[
            pltpu.VMEM((2, N_NEIGHBOR * ECH), i32),
            pltpu.VMEM((2, ECH, N_NEIGHBOR), i32),
            pltpu.SemaphoreType.DMA,
            pltpu.SemaphoreType.DMA,
            pltpu.SemaphoreType.DMA,
            pltpu.SemaphoreType.DMA,
        ],
    )
    ent_rm, rel_rm = relayout(entt, relt)

    expand = pl.kernel(
        _rf_body,
        out_type=(
            jax.ShapeDtypeStruct((N_NEIGHBOR, BATCH), i32),
            jax.ShapeDtypeStruct((BATCH, WIDE), i32),
            jax.ShapeDtypeStruct((N_NEIGHBOR, BATCH), i32),
            jax.ShapeDtypeStruct((BATCH, WIDE), i32),
        ),
        mesh=mesh,
        compiler_params=params,
        scratch_types=[
            pltpu.VMEM((SPW,), i32),
            pltpu.VMEM((SPW, N_NEIGHBOR), i32),
            pltpu.VMEM((SPW, N_NEIGHBOR), i32),
            pltpu.VMEM((N_NEIGHBOR, SPW), i32),
            pltpu.VMEM((N_NEIGHBOR, SPW), i32),
            pltpu.VMEM((2, SPW, N_NEIGHBOR), i32),
            pltpu.VMEM((2, SPW, N_NEIGHBOR), i32),
            pltpu.SemaphoreType.DMA,
            pltpu.SemaphoreType.DMA,
            pltpu.SemaphoreType.DMA,
            pltpu.SemaphoreType.DMA,
            pltpu.SemaphoreType.DMA,
            pltpu.SemaphoreType.DMA,
        ],
    )
    ent1t, ent2, rel1t, rel2 = expand(x_flat, ent_rm, rel_rm)
    return (x, ent1t.T, ent2, rel1t.T, rel2)
